# baseline (speedup 1.0000x reference)
import jax
import jax.numpy as jnp
from jax import lax
from jax.experimental import pallas as pl
from jax.experimental.pallas import tpu as pltpu

BSIZE, NQUERY, SEQL, ISIZE = 32, 16, 8192, 128
NROWS = BSIZE * NQUERY
CBLK = 32
NCHUNK = NROWS // CBLK
NBUF = 6


def _stream_copy(x_ref, bufs, ssem, c):
    return pltpu.make_async_copy(
        x_ref.at[pl.ds(c * CBLK, CBLK), :],
        bufs.at[c % NBUF],
        ssem.at[c % NBUF],
    )


def _fused_body(x_ref, vmat_ref, out_ref, bufs, idx_vmem, idx_smem,
                rows_vmem, ssem, isem, gsem, osem):

    def _issue_gathers(chunk):
        pltpu.make_async_copy(
            idx_vmem.at[pl.ds(chunk * CBLK, CBLK), :],
            idx_smem.at[pl.ds(chunk * CBLK, CBLK), :],
            isem,
        ).wait()
        for r in range(CBLK):
            row = chunk * CBLK + r
            s = idx_smem[row, 0]
            pltpu.make_async_copy(
                vmat_ref.at[pl.ds(s, 1), :],
                rows_vmem.at[pl.ds(row, 1), :],
                gsem,
            ).start()

    for c in range(NBUF):
        _stream_copy(x_ref, bufs, ssem, c).start()

    for c in range(NCHUNK):
        _stream_copy(x_ref, bufs, ssem, c).wait()
        x = bufs[c % NBUF]
        m = jnp.max(x, axis=1, keepdims=True)
        col = lax.broadcasted_iota(jnp.int32, x.shape, 1)
        idx = jnp.min(jnp.where(x == m, col, jnp.int32(SEQL)), axis=1,
                      keepdims=True)
        rows = c * CBLK + lax.broadcasted_iota(jnp.int32, (CBLK, 1), 0)
        flat = idx + (rows // NQUERY) * SEQL
        idx_vmem[pl.ds(c * CBLK, CBLK), :] = flat
        pltpu.make_async_copy(
            idx_vmem.at[pl.ds(c * CBLK, CBLK), :],
            idx_smem.at[pl.ds(c * CBLK, CBLK), :],
            isem,
        ).start()
        if c + NBUF < NCHUNK:
            _stream_copy(x_ref, bufs, ssem, c + NBUF).start()
        if c > 0:
            _issue_gathers(c - 1)

    _issue_gathers(NCHUNK - 1)
    pltpu.make_async_copy(
        vmat_ref.at[pl.ds(0, NROWS), :], rows_vmem, gsem,
    ).wait()
    pltpu.make_async_copy(rows_vmem, out_ref, osem).start()
    pltpu.make_async_copy(rows_vmem, out_ref, osem).wait()


_fused_call = pl.pallas_call(
    _fused_body,
    in_specs=[
        pl.BlockSpec(memory_space=pltpu.MemorySpace.HBM),
        pl.BlockSpec(memory_space=pltpu.MemorySpace.HBM),
    ],
    out_specs=pl.BlockSpec(memory_space=pltpu.MemorySpace.HBM),
    out_shape=jax.ShapeDtypeStruct((NROWS, ISIZE), jnp.float32),
    scratch_shapes=[
        pltpu.VMEM((NBUF, CBLK, SEQL), jnp.float32),
        pltpu.VMEM((NROWS, 1), jnp.int32),
        pltpu.SMEM((NROWS, 1), jnp.int32),
        pltpu.VMEM((NROWS, ISIZE), jnp.float32),
        pltpu.SemaphoreType.DMA((NBUF,)),
        pltpu.SemaphoreType.DMA,
        pltpu.SemaphoreType.DMA,
        pltpu.SemaphoreType.DMA,
    ],
)


def kernel(attnmat, vmat):
    bsize, nquery, seql = attnmat.shape
    isize = vmat.shape[-1]
    attn2d = attnmat.reshape(bsize * nquery, seql)
    flat_v = vmat.reshape(bsize * seql, isize)
    out = _fused_call(attn2d, flat_v)
    return out.reshape(bsize, nquery, isize)


# speedup vs baseline: 1.2763x; 1.2763x over previous
import jax
import jax.numpy as jnp
from jax import lax
from jax.experimental import pallas as pl
from jax.experimental.pallas import tpu as pltpu

BSIZE, NQUERY, SEQL, ISIZE = 32, 16, 8192, 128
NROWS = BSIZE * NQUERY
CBLK = 128
NCHUNK = NROWS // CBLK
NBUF = 4


def _stream_copy(x_ref, bufs, ssem, c):
    return pltpu.make_async_copy(
        x_ref.at[pl.ds(c * CBLK, CBLK), :],
        bufs.at[c % NBUF],
        ssem.at[c % NBUF],
    )


def _fused_body(x_ref, vmat_ref, out_ref, bufs, idx_vmem, idx_smem,
                rows_vmem, ssem, isem, gsem, osem):

    def _issue_gathers(chunk):
        pltpu.make_async_copy(
            idx_vmem.at[pl.ds(chunk * CBLK, CBLK), :],
            idx_smem.at[pl.ds(chunk * CBLK, CBLK), :],
            isem,
        ).wait()
        for r in range(CBLK):
            row = chunk * CBLK + r
            s = idx_smem[row, 0]
            pltpu.make_async_copy(
                vmat_ref.at[pl.ds(s, 1), :],
                rows_vmem.at[pl.ds(row, 1), :],
                gsem,
            ).start()

    for c in range(NBUF):
        _stream_copy(x_ref, bufs, ssem, c).start()

    for c in range(NCHUNK):
        _stream_copy(x_ref, bufs, ssem, c).wait()
        x = bufs[c % NBUF]
        m = jnp.max(x, axis=1, keepdims=True)
        col = lax.broadcasted_iota(jnp.int32, x.shape, 1)
        idx = jnp.min(jnp.where(x == m, col, jnp.int32(SEQL)), axis=1,
                      keepdims=True)
        rows = c * CBLK + lax.broadcasted_iota(jnp.int32, (CBLK, 1), 0)
        flat = idx + (rows // NQUERY) * SEQL
        idx_vmem[pl.ds(c * CBLK, CBLK), :] = flat
        pltpu.make_async_copy(
            idx_vmem.at[pl.ds(c * CBLK, CBLK), :],
            idx_smem.at[pl.ds(c * CBLK, CBLK), :],
            isem,
        ).start()
        if c + NBUF < NCHUNK:
            _stream_copy(x_ref, bufs, ssem, c + NBUF).start()
        if c > 0:
            _issue_gathers(c - 1)

    _issue_gathers(NCHUNK - 1)
    pltpu.make_async_copy(
        vmat_ref.at[pl.ds(0, NROWS), :], rows_vmem, gsem,
    ).wait()
    pltpu.make_async_copy(rows_vmem, out_ref, osem).start()
    pltpu.make_async_copy(rows_vmem, out_ref, osem).wait()


_fused_call = pl.pallas_call(
    _fused_body,
    in_specs=[
        pl.BlockSpec(memory_space=pltpu.MemorySpace.HBM),
        pl.BlockSpec(memory_space=pltpu.MemorySpace.HBM),
    ],
    out_specs=pl.BlockSpec(memory_space=pltpu.MemorySpace.HBM),
    out_shape=jax.ShapeDtypeStruct((NROWS, ISIZE), jnp.float32),
    scratch_shapes=[
        pltpu.VMEM((NBUF, CBLK, SEQL), jnp.float32),
        pltpu.VMEM((NROWS, 1), jnp.int32),
        pltpu.SMEM((NROWS, 1), jnp.int32),
        pltpu.VMEM((NROWS, ISIZE), jnp.float32),
        pltpu.SemaphoreType.DMA((NBUF,)),
        pltpu.SemaphoreType.DMA,
        pltpu.SemaphoreType.DMA,
        pltpu.SemaphoreType.DMA,
    ],
)


def kernel(attnmat, vmat):
    bsize, nquery, seql = attnmat.shape
    isize = vmat.shape[-1]
    attn2d = attnmat.reshape(bsize * nquery, seql)
    flat_v = vmat.reshape(bsize * seql, isize)
    out = _fused_call(attn2d, flat_v)
    return out.reshape(bsize, nquery, isize)
